# baseline (speedup 1.0000x reference)
import functools

import jax
import jax.numpy as jnp
from jax import lax
from jax.experimental import pallas as pl
from jax.experimental.pallas import tpu as pltpu
from jax.experimental.pallas import tpu_sc as plsc

NUM_BUCKETS = 32
NUM_HEADS = 16
SEQ = 2048
WIDTH = 2 * SEQ
NSKEW = 8
THRESH = (12, 16, 23, 32, 46, 64, 91)
NC, NS, LANES = 2, 16, 16


def _bucket_of(d):
    ad = jnp.abs(d)
    zero = jnp.zeros_like(d)
    one = jnp.ones_like(d)
    large = jnp.full_like(d, 8)
    for t in THRESH:
        large = large + jnp.minimum(jnp.maximum(ad - (t - 1), zero), one)
    b = jnp.minimum(ad, large)
    return b + 16 * jnp.minimum(jnp.maximum(d, zero), one)


def _build_wskew_sc(bias_table):
    rows = NUM_HEADS * NSKEW
    nworkers = NC * NS
    rows_per_worker = rows // nworkers
    mesh = plsc.VectorSubcoreMesh(
        core_axis_name="c", subcore_axis_name="s", num_cores=NC, num_subcores=NS
    )

    @functools.partial(
        pl.kernel,
        mesh=mesh,
        out_type=jax.ShapeDtypeStruct((NUM_HEADS, NSKEW, WIDTH), jnp.float32),
        scratch_types=[
            pltpu.VMEM((NUM_BUCKETS * NUM_HEADS,), jnp.float32),
            pltpu.VMEM((WIDTH,), jnp.float32),
        ],
        compiler_params=pltpu.CompilerParams(needs_layout_passes=False),
    )
    def sc_kernel(table_hbm, out_hbm, table_v, row_v):
        wid = lax.axis_index("s") * NC + lax.axis_index("c")
        pltpu.sync_copy(table_hbm, table_v)
        for j in range(rows_per_worker):
            r = wid * rows_per_worker + j
            h = r // NSKEW
            i = r % NSKEW

            def chunk(c, carry):
                t = lax.iota(jnp.int32, LANES) + c * LANES
                d = t - i - SEQ
                idx = _bucket_of(d) * NUM_HEADS + h
                row_v[pl.ds(c * LANES, LANES)] = plsc.load_gather(table_v, [idx])
                return carry

            lax.fori_loop(0, WIDTH // LANES, chunk, 0)
            pltpu.sync_copy(row_v, out_hbm.at[h, i])

    return sc_kernel(bias_table.reshape(-1))


def _add_bias_tc(scores, wskew, block_q=256):
    _, heads, seq_q, seq_k = scores.shape
    nkv = seq_k // 128
    grid = (heads, seq_q // block_q)
    s5 = scores.reshape(1, heads, seq_q, nkv, 128)
    w4 = wskew.reshape(NUM_HEADS, NSKEW, WIDTH // 128, 128)

    def body(s_ref, w_ref, o_ref):
        qb = pl.program_id(1)
        for g in range(block_q // NSKEW):
            r = (-NSKEW * g) % 128
            cterm = (NSKEW * g + r) // 128
            c = (SEQ // 128) - (block_q // 128) * qb - cterm
            if r == 0:
                tile = w_ref[0, :, pl.ds(c, nkv), :]
            else:
                a = w_ref[0, :, pl.ds(c, nkv + 1), :]
                rolled = pltpu.roll(a, 128 - r, axis=2)
                lane = lax.broadcasted_iota(jnp.int32, (NSKEW, nkv, 128), 2)
                tile = jnp.where(lane < 128 - r, rolled[:, :nkv, :], rolled[:, 1:, :])
            sl = slice(g * NSKEW, (g + 1) * NSKEW)
            o_ref[0, 0, sl, :, :] = s_ref[0, 0, sl, :, :] + tile

    out = pl.pallas_call(
        body,
        grid=grid,
        in_specs=[
            pl.BlockSpec((1, 1, block_q, nkv, 128), lambda h, q: (0, h, q, 0, 0)),
            pl.BlockSpec((1, NSKEW, WIDTH // 128, 128), lambda h, q: (h, 0, 0, 0)),
        ],
        out_specs=pl.BlockSpec((1, 1, block_q, nkv, 128), lambda h, q: (0, h, q, 0, 0)),
        out_shape=jax.ShapeDtypeStruct(s5.shape, scores.dtype),
    )(s5, w4)
    return out.reshape(scores.shape)


@jax.jit
def kernel(x, attention_scores, bias_table):
    del x
    wskew = _build_wskew_sc(bias_table)
    return _add_bias_tc(attention_scores, wskew)


# speedup vs baseline: 3.0090x; 3.0090x over previous
import functools

import jax
import jax.numpy as jnp
from jax import lax
from jax.experimental import pallas as pl
from jax.experimental.pallas import tpu as pltpu
from jax.experimental.pallas import tpu_sc as plsc

NUM_BUCKETS = 32
NUM_HEADS = 16
SEQ = 2048
WIDTH = 2 * SEQ
NSKEW = 8
THRESH = (12, 16, 23, 32, 46, 64, 91)
NC, NS, LANES = 2, 16, 16


def _bucket_of(d):
    ad = jnp.abs(d)
    zero = jnp.zeros_like(d)
    one = jnp.ones_like(d)
    large = jnp.full_like(d, 8)
    for t in THRESH:
        large = large + jnp.minimum(jnp.maximum(ad - (t - 1), zero), one)
    b = jnp.minimum(ad, large)
    return b + 16 * jnp.minimum(jnp.maximum(d, zero), one)


def _build_wskew_sc(bias_table):
    rows = NUM_HEADS * NSKEW
    nworkers = NC * NS
    rows_per_worker = rows // nworkers
    mesh = plsc.VectorSubcoreMesh(
        core_axis_name="c", subcore_axis_name="s", num_cores=NC, num_subcores=NS
    )

    @functools.partial(
        pl.kernel,
        mesh=mesh,
        out_type=jax.ShapeDtypeStruct((NUM_HEADS, NSKEW, WIDTH), jnp.float32),
        scratch_types=[
            pltpu.VMEM((NUM_BUCKETS * NUM_HEADS,), jnp.float32),
            pltpu.VMEM((WIDTH,), jnp.float32),
        ],
        compiler_params=pltpu.CompilerParams(needs_layout_passes=False),
    )
    def sc_kernel(table_hbm, out_hbm, table_v, row_v):
        wid = lax.axis_index("s") * NC + lax.axis_index("c")
        pltpu.sync_copy(table_hbm, table_v)
        for j in range(rows_per_worker):
            r = wid * rows_per_worker + j
            h = r // NSKEW
            i = r % NSKEW

            def chunk(c, carry):
                t = lax.iota(jnp.int32, LANES) + c * LANES
                d = t - i - SEQ
                idx = _bucket_of(d) * NUM_HEADS + h
                row_v[pl.ds(c * LANES, LANES)] = plsc.load_gather(table_v, [idx])
                return carry

            lax.fori_loop(0, WIDTH // LANES, chunk, 0)
            pltpu.sync_copy(row_v, out_hbm.at[h, i])

    return sc_kernel(bias_table.reshape(-1))


def _add_bias_tc(scores, wskew, block_q=256):
    _, heads, seq_q, seq_k = scores.shape
    grid = (heads, seq_q // block_q)

    def body(s_ref, w_ref, o_ref):
        qb = pl.program_id(1)
        for g in range(block_q // NSKEW):
            r = (-NSKEW * g) % 128
            cterm = (NSKEW * g + r) // 128
            c = (SEQ // 128) - (block_q // 128) * qb - cterm
            if r == 0:
                tile = w_ref[0, :, pl.ds(c * 128, seq_k)]
            else:
                a = w_ref[0, :, pl.ds(c * 128, seq_k + 128)]
                tile = pltpu.roll(a, seq_k + 128 - r, axis=1)[:, :seq_k]
            sl = slice(g * NSKEW, (g + 1) * NSKEW)
            o_ref[0, 0, sl, :] = s_ref[0, 0, sl, :] + tile

    return pl.pallas_call(
        body,
        grid=grid,
        in_specs=[
            pl.BlockSpec((1, 1, block_q, seq_k), lambda h, q: (0, h, q, 0)),
            pl.BlockSpec((1, NSKEW, WIDTH), lambda h, q: (h, 0, 0)),
        ],
        out_specs=pl.BlockSpec((1, 1, block_q, seq_k), lambda h, q: (0, h, q, 0)),
        out_shape=jax.ShapeDtypeStruct(scores.shape, scores.dtype),
    )(scores, wskew)


@jax.jit
def kernel(x, attention_scores, bias_table):
    del x
    wskew = _build_wskew_sc(bias_table)
    return _add_bias_tc(attention_scores, wskew)


# speedup vs baseline: 3.4523x; 1.1473x over previous
import functools

import jax
import jax.numpy as jnp
from jax import lax
from jax.experimental import pallas as pl
from jax.experimental.pallas import tpu as pltpu
from jax.experimental.pallas import tpu_sc as plsc

NUM_BUCKETS = 32
NUM_HEADS = 16
SEQ = 2048
WIDTH = 2 * SEQ
NSKEW = 8
THRESH = (12, 16, 23, 32, 46, 64, 91)
NC, NS, LANES = 2, 16, 16


def _bucket_of(d):
    ad = jnp.abs(d)
    zero = jnp.zeros_like(d)
    one = jnp.ones_like(d)
    large = jnp.full_like(d, 8)
    for t in THRESH:
        large = large + jnp.minimum(jnp.maximum(ad - (t - 1), zero), one)
    b = jnp.minimum(ad, large)
    return b + 16 * jnp.minimum(jnp.maximum(d, zero), one)


def _build_wskew_sc(bias_table):
    rows = NUM_HEADS * NSKEW
    nworkers = NC * NS
    rows_per_worker = rows // nworkers
    mesh = plsc.VectorSubcoreMesh(
        core_axis_name="c", subcore_axis_name="s", num_cores=NC, num_subcores=NS
    )

    @functools.partial(
        pl.kernel,
        mesh=mesh,
        out_type=jax.ShapeDtypeStruct((NUM_HEADS, NSKEW, WIDTH), jnp.float32),
        scratch_types=[
            pltpu.VMEM((NUM_BUCKETS * NUM_HEADS,), jnp.float32),
            pltpu.VMEM((WIDTH,), jnp.float32),
        ],
        compiler_params=pltpu.CompilerParams(needs_layout_passes=False),
    )
    def sc_kernel(table_hbm, out_hbm, table_v, row_v):
        wid = lax.axis_index("s") * NC + lax.axis_index("c")
        pltpu.sync_copy(table_hbm, table_v)
        for j in range(rows_per_worker):
            r = wid * rows_per_worker + j
            h = r // NSKEW
            i = r % NSKEW

            def chunk(c, carry):
                t = lax.iota(jnp.int32, LANES) + c * LANES
                d = t - i - SEQ
                idx = _bucket_of(d) * NUM_HEADS + h
                row_v[pl.ds(c * LANES, LANES)] = plsc.load_gather(table_v, [idx])
                return carry

            lax.fori_loop(0, WIDTH // LANES, chunk, 0)
            pltpu.sync_copy(row_v, out_hbm.at[h, i])

    return sc_kernel(bias_table.reshape(-1))


def _add_bias_tc(scores, wskew, block_q=512):
    _, heads, seq_q, seq_k = scores.shape
    grid = (heads, seq_q // block_q)

    def body(s_ref, w_ref, o_ref):
        qb = pl.program_id(1)
        for g in range(block_q // NSKEW):
            r = (-NSKEW * g) % 128
            cterm = (NSKEW * g + r) // 128
            c = (SEQ // 128) - (block_q // 128) * qb - cterm
            if r == 0:
                tile = w_ref[0, :, pl.ds(c * 128, seq_k)]
            else:
                a = w_ref[0, :, pl.ds(c * 128, seq_k + 128)]
                tile = pltpu.roll(a, seq_k + 128 - r, axis=1)[:, :seq_k]
            sl = slice(g * NSKEW, (g + 1) * NSKEW)
            o_ref[0, 0, sl, :] = s_ref[0, 0, sl, :] + tile

    return pl.pallas_call(
        body,
        grid=grid,
        in_specs=[
            pl.BlockSpec((1, 1, block_q, seq_k), lambda h, q: (0, h, q, 0)),
            pl.BlockSpec((1, NSKEW, WIDTH), lambda h, q: (h, 0, 0)),
        ],
        out_specs=pl.BlockSpec((1, 1, block_q, seq_k), lambda h, q: (0, h, q, 0)),
        out_shape=jax.ShapeDtypeStruct(scores.shape, scores.dtype),
    )(scores, wskew)


@jax.jit
def kernel(x, attention_scores, bias_table):
    del x
    wskew = _build_wskew_sc(bias_table)
    return _add_bias_tc(attention_scores, wskew)


# speedup vs baseline: 3.5377x; 1.0247x over previous
import functools

import jax
import jax.numpy as jnp
from jax import lax
from jax.experimental import pallas as pl
from jax.experimental.pallas import tpu as pltpu
from jax.experimental.pallas import tpu_sc as plsc

NUM_BUCKETS = 32
NUM_HEADS = 16
SEQ = 2048
WIDTH = 2 * SEQ
NSKEW = 8
THRESH = (12, 16, 23, 32, 46, 64, 91)
NC, NS, LANES = 2, 16, 16


def _bucket_of(d):
    ad = jnp.abs(d)
    zero = jnp.zeros_like(d)
    one = jnp.ones_like(d)
    large = jnp.full_like(d, 8)
    for t in THRESH:
        large = large + jnp.minimum(jnp.maximum(ad - (t - 1), zero), one)
    b = jnp.minimum(ad, large)
    return b + 16 * jnp.minimum(jnp.maximum(d, zero), one)


def _build_wskew_sc(bias_table):
    rows = NUM_HEADS * NSKEW
    nworkers = NC * NS
    rows_per_worker = rows // nworkers
    mesh = plsc.VectorSubcoreMesh(
        core_axis_name="c", subcore_axis_name="s", num_cores=NC, num_subcores=NS
    )

    @functools.partial(
        pl.kernel,
        mesh=mesh,
        out_type=jax.ShapeDtypeStruct((NUM_HEADS, NSKEW, WIDTH), jnp.float32),
        scratch_types=[
            pltpu.VMEM((NUM_BUCKETS * NUM_HEADS,), jnp.float32),
            pltpu.VMEM((WIDTH,), jnp.float32),
        ],
        compiler_params=pltpu.CompilerParams(needs_layout_passes=False),
    )
    def sc_kernel(table_hbm, out_hbm, table_v, row_v):
        wid = lax.axis_index("s") * NC + lax.axis_index("c")
        pltpu.sync_copy(table_hbm, table_v)
        for j in range(rows_per_worker):
            r = wid * rows_per_worker + j
            h = r // NSKEW
            i = r % NSKEW

            def chunk(c, carry):
                t = lax.iota(jnp.int32, LANES) + c * LANES
                d = t - i - SEQ
                idx = _bucket_of(d) * NUM_HEADS + h
                row_v[pl.ds(c * LANES, LANES)] = plsc.load_gather(table_v, [idx])
                return carry

            lax.fori_loop(0, WIDTH // LANES, chunk, 0)
            pltpu.sync_copy(row_v, out_hbm.at[h, i])

    return sc_kernel(bias_table.reshape(-1))


def _add_bias_tc(scores, wskew, block_q=1024):
    _, heads, seq_q, seq_k = scores.shape
    grid = (heads, seq_q // block_q)

    def body(s_ref, w_ref, o_ref):
        qb = pl.program_id(1)
        for g in range(block_q // NSKEW):
            r = (-NSKEW * g) % 128
            cterm = (NSKEW * g + r) // 128
            c = (SEQ // 128) - (block_q // 128) * qb - cterm
            if r == 0:
                tile = w_ref[0, :, pl.ds(c * 128, seq_k)]
            else:
                a = w_ref[0, :, pl.ds(c * 128, seq_k + 128)]
                tile = pltpu.roll(a, seq_k + 128 - r, axis=1)[:, :seq_k]
            sl = slice(g * NSKEW, (g + 1) * NSKEW)
            o_ref[0, 0, sl, :] = s_ref[0, 0, sl, :] + tile

    return pl.pallas_call(
        body,
        grid=grid,
        in_specs=[
            pl.BlockSpec((1, 1, block_q, seq_k), lambda h, q: (0, h, q, 0)),
            pl.BlockSpec((1, NSKEW, WIDTH), lambda h, q: (h, 0, 0)),
        ],
        out_specs=pl.BlockSpec((1, 1, block_q, seq_k), lambda h, q: (0, h, q, 0)),
        out_shape=jax.ShapeDtypeStruct(scores.shape, scores.dtype),
    )(scores, wskew)


@jax.jit
def kernel(x, attention_scores, bias_table):
    del x
    wskew = _build_wskew_sc(bias_table)
    return _add_bias_tc(attention_scores, wskew)


# speedup vs baseline: 3.6764x; 1.0392x over previous
import functools

import jax
import jax.numpy as jnp
from jax import lax
from jax.experimental import pallas as pl
from jax.experimental.pallas import tpu as pltpu
from jax.experimental.pallas import tpu_sc as plsc

NUM_BUCKETS = 32
NUM_HEADS = 16
SEQ = 2048
WIDTH = 2 * SEQ
NSKEW = 8
THRESH = (12, 16, 23, 32, 46, 64, 91)
NC, NS, LANES = 2, 16, 16


def _bucket_of(d):
    ad = jnp.abs(d)
    zero = jnp.zeros_like(d)
    one = jnp.ones_like(d)
    large = jnp.full_like(d, 8)
    for t in THRESH:
        large = large + jnp.minimum(jnp.maximum(ad - (t - 1), zero), one)
    b = jnp.minimum(ad, large)
    return b + 16 * jnp.minimum(jnp.maximum(d, zero), one)


def _build_w_sc(bias_table):
    mesh = plsc.VectorSubcoreMesh(
        core_axis_name="c", subcore_axis_name="s", num_cores=NC, num_subcores=NS
    )

    @functools.partial(
        pl.kernel,
        mesh=mesh,
        out_type=jax.ShapeDtypeStruct((NUM_HEADS * WIDTH,), jnp.float32),
        scratch_types=[
            pltpu.VMEM((NUM_BUCKETS * NUM_HEADS,), jnp.float32),
            pltpu.VMEM((SEQ,), jnp.float32),
        ],
        compiler_params=pltpu.CompilerParams(needs_layout_passes=False),
    )
    def sc_kernel(table_hbm, out_hbm, table_v, half_v):
        wid = lax.axis_index("s") * NC + lax.axis_index("c")
        h = wid // 2
        half = wid % 2
        pltpu.sync_copy(table_hbm, table_v)

        def chunk(c, carry):
            j = lax.iota(jnp.int32, LANES) + c * LANES + half * SEQ
            idx = _bucket_of(j - SEQ) * NUM_HEADS + h
            half_v[pl.ds(c * LANES, LANES)] = plsc.load_gather(table_v, [idx])
            return carry

        lax.fori_loop(0, SEQ // LANES, chunk, 0)
        pltpu.sync_copy(half_v, out_hbm.at[pl.ds(h * WIDTH + half * SEQ, SEQ)])

    return sc_kernel(bias_table.reshape(-1)).reshape(NUM_HEADS, WIDTH)


def _add_bias_tc(scores, w, block_q=1024):
    _, heads, seq_q, seq_k = scores.shape
    grid = (heads, seq_q // block_q)
    w3 = w.reshape(heads, 1, WIDTH)

    def body(s_ref, w_ref, o_ref, wskew_ref):
        qb = pl.program_id(1)

        @pl.when(qb == 0)
        def _prologue():
            for i in range(NSKEW):
                wskew_ref[i : i + 1, :] = pltpu.roll(w_ref[0], i, axis=1)

        for g in range(block_q // NSKEW):
            r = (-NSKEW * g) % 128
            cterm = (NSKEW * g + r) // 128
            c = (SEQ // 128) - (block_q // 128) * qb - cterm
            if r == 0:
                tile = wskew_ref[:, pl.ds(c * 128, seq_k)]
            else:
                a = wskew_ref[:, pl.ds(c * 128, seq_k + 128)]
                tile = pltpu.roll(a, seq_k + 128 - r, axis=1)[:, :seq_k]
            sl = slice(g * NSKEW, (g + 1) * NSKEW)
            o_ref[0, 0, sl, :] = s_ref[0, 0, sl, :] + tile

    return pl.pallas_call(
        body,
        grid=grid,
        in_specs=[
            pl.BlockSpec((1, 1, block_q, seq_k), lambda h, q: (0, h, q, 0)),
            pl.BlockSpec((1, 1, WIDTH), lambda h, q: (h, 0, 0)),
        ],
        out_specs=pl.BlockSpec((1, 1, block_q, seq_k), lambda h, q: (0, h, q, 0)),
        out_shape=jax.ShapeDtypeStruct(scores.shape, scores.dtype),
        scratch_shapes=[pltpu.VMEM((NSKEW, WIDTH), jnp.float32)],
    )(scores, w3)


@jax.jit
def kernel(x, attention_scores, bias_table):
    del x
    w = _build_w_sc(bias_table)
    return _add_bias_tc(attention_scores, w)
